# baseline (speedup 1.0000x reference)
import jax
import jax.numpy as jnp
from jax import lax
from jax.experimental import pallas as pl
from jax.experimental.pallas import tpu as pltpu
from jax.experimental.pallas import tpu_sc as plsc

_N = 10000
_E = 320000
_D = 128
_NW = 32
_EPT = _E // _NW
_G = 16
_SLAB = 2000
_NSLAB = _EPT // _SLAB
_NG = _SLAB // _G
_RPT = _N // 16
_BP = 1000


def _pre_body(x_ref, wc_ref, wp_ref, wv_ref, p_ref, pc_ref):
    x = x_ref[...]
    xc = jnp.dot(x, wc_ref[...].T, preferred_element_type=jnp.float32)
    h = jnp.dot(xc, wp_ref[...].T, preferred_element_type=jnp.float32)
    v = jnp.dot(x, wv_ref[...].T, preferred_element_type=jnp.float32)
    p_ref[:, 0:_D] = h.astype(jnp.bfloat16)
    p_ref[:, _D:2 * _D] = v.astype(jnp.bfloat16)
    pc_ref[...] = jnp.tanh(jnp.sqrt(jnp.sum(xc * xc, axis=1, keepdims=True)))


def _post_body(x_ref, z_ref, acc_ref, sd_ref, pc_ref, o_ref):
    s2 = acc_ref[0] + acc_ref[1]
    sd = jnp.sum(sd_ref[...], axis=0)
    s1 = sd[:, 0:1]
    deg = sd[:, 1:2]
    f = jnp.where(deg > 0.0, 1.0 / (deg * pc_ref[...]), 0.0)
    o_ref[...] = x_ref[...] - f * (s1 * z_ref[...] - s2)


def _edge_body(p_hbm, z_hbm, row_hbm, col_hbm, ew_hbm, acc_hbm, sd_hbm,
               row_v, col_v, ew_v, idx_r0, idx_c0, idx_r1, idx_c1,
               idx_rc0, idx_rc1, trow0, trow1, val0, val1,
               s1d, spacc, sem_a, sem_b, sem_s0, sem_s1):
    cid = lax.axis_index("c")
    sid = lax.axis_index("s")
    wid = sid * 2 + cid
    ebase = wid * _EPT

    zeros16 = jnp.zeros((16,), jnp.float32)
    for r in range(_G):
        for j in range(_D // 16):
            val0[r, pl.ds(16 * j, 16)] = zeros16
            val1[r, pl.ds(16 * j, 16)] = zeros16
    izero = jnp.zeros((16,), jnp.int32)
    idx_r0[...] = izero
    idx_r1[...] = izero
    def zclr(i, carry):
        s1d[pl.ds(i * 16, 16)] = zeros16
        return carry
    lax.fori_loop(0, 2 * _N // 16, zclr, 0)
    n_units = _N // _G
    for t in range(pl.cdiv(n_units, 16)):
        u = sid + 16 * t

        @pl.when(u < n_units)
        def _zero():
            pltpu.sync_copy(val0, spacc.at[pl.ds(u * _G, _G)])
    plsc.subcore_barrier()
    pltpu.async_copy(val0, spacc.at[idx_r0], sem_s0, add=True)
    pltpu.async_copy(val1, spacc.at[idx_r1], sem_s1, add=True)

    def issue(g, idx_r, idx_c, idx_rc, trow, val, sem, sem_s):
        pltpu.make_async_copy(val, spacc.at[idx_r], sem_s).wait()
        r16 = row_v[pl.ds(g * _G, _G)]
        c16 = col_v[pl.ds(g * _G, _G)]
        idx_r[...] = r16
        idx_c[...] = c16
        idx_rc[pl.ds(0, _G)] = r16
        idx_rc[pl.ds(_G, _G)] = c16
        pltpu.async_copy(p_hbm.at[idx_rc], trow, sem)
        pltpu.async_copy(z_hbm.at[idx_c], val, sem)

    def compute(g, idx_r, idx_c, idx_rc, trow, sem, val, sem_s):
        pltpu.make_async_copy(p_hbm.at[idx_rc], trow, sem).wait()
        pltpu.make_async_copy(z_hbm.at[idx_c], val, sem).wait()
        lane = lax.iota(jnp.int32, 16)

        def edot(e, dhdv):
            dh, dv = dhdv
            ph = []
            pv = []
            for j in range(_D // 32):
                a1, b1 = plsc.unpack(
                    plsc.bitcast(trow[e, pl.ds(16 * j, 16)], jnp.bfloat16),
                    format=plsc.PackFormat.INTERLEAVED)
                a2, b2 = plsc.unpack(
                    plsc.bitcast(trow[e + _G, pl.ds(16 * j, 16)], jnp.bfloat16),
                    format=plsc.PackFormat.INTERLEAVED)
                ph.append(a1 * a2)
                ph.append(b1 * b2)
            for j in range(_D // 32, _D // 16):
                a1, b1 = plsc.unpack(
                    plsc.bitcast(trow[e, pl.ds(16 * j, 16)], jnp.bfloat16),
                    format=plsc.PackFormat.INTERLEAVED)
                a2, b2 = plsc.unpack(
                    plsc.bitcast(trow[e + _G, pl.ds(16 * j, 16)], jnp.bfloat16),
                    format=plsc.PackFormat.INTERLEAVED)
                d0 = a1 - a2
                d1 = b1 - b2
                pv.append(d0 * d0)
                pv.append(d1 * d1)
            while len(ph) > 1:
                ph = [ph[k] + ph[k + 1] for k in range(0, len(ph), 2)]
                pv = [pv[k] + pv[k + 1] for k in range(0, len(pv), 2)]
            acc = ph[0]
            acc2 = pv[0]
            for s in (8, 4, 2, 1):
                perm = jnp.bitwise_xor(lane, s)
                acc = acc + jnp.take(acc, perm)
                acc2 = acc2 + jnp.take(acc2, perm)
            dh = jnp.where(lane == e, acc, dh)
            dv = jnp.where(lane == e, acc2, dv)
            return dh, dv

        dh, dv = lax.fori_loop(
            0, _G, edot,
            (jnp.zeros((16,), jnp.float32), jnp.zeros((16,), jnp.float32)))
        ewg = ew_v[pl.ds(g * _G, _G)]
        e2 = jnp.exp(jnp.abs(dh) * 2.0)
        t1 = 1.0 - 2.0 / (e2 + 1.0)
        iy = jnp.int32(0x5F3759DF) - lax.shift_right_arithmetic(
            plsc.bitcast(dv, jnp.int32), 1)
        y = plsc.bitcast(iy, jnp.float32)
        for _ in range(3):
            y = y * (1.5 - 0.5 * dv * y * y)
        nv = dv * y
        r2 = 2.0 / (nv + 1e-6)
        e2b = jnp.exp(r2)
        t2 = 1.0 - 2.0 / (e2b + 1.0)
        ug = ewg * t1 * t2
        r2v = idx_r[...] * 2
        plsc.addupdate_scatter(s1d, [r2v], ug)
        plsc.addupdate_scatter(s1d, [r2v + 1], ewg)
        for e in range(_G):
            u = ug[e]
            for j in range(_D // 16):
                val[e, pl.ds(16 * j, 16)] = val[e, pl.ds(16 * j, 16)] * u
        pltpu.async_copy(val, spacc.at[idx_r], sem_s, add=True)

    def pair(gg, carry):
        g0 = gg * 2
        issue(g0 + 1, idx_r1, idx_c1, idx_rc1, trow1, val1, sem_b, sem_s1)
        compute(g0, idx_r0, idx_c0, idx_rc0, trow0, sem_a, val0, sem_s0)

        @pl.when(gg < _NG // 2 - 1)
        def _prefetch():
            issue(g0 + 2, idx_r0, idx_c0, idx_rc0, trow0, val0, sem_a, sem_s0)
        compute(g0 + 1, idx_r1, idx_c1, idx_rc1, trow1, sem_b, val1, sem_s1)
        return carry

    for slab in range(_NSLAB):
        sbase = ebase + slab * _SLAB
        pltpu.sync_copy(row_hbm.at[pl.ds(sbase, _SLAB)], row_v)
        pltpu.sync_copy(col_hbm.at[pl.ds(sbase, _SLAB)], col_v)
        pltpu.sync_copy(ew_hbm.at[pl.ds(sbase, _SLAB)], ew_v)
        issue(0, idx_r0, idx_c0, idx_rc0, trow0, val0, sem_a, sem_s0)
        lax.fori_loop(0, _NG // 2, pair, 0)
        issue(_NG - 1, idx_r0, idx_c0, idx_rc0, trow0, val0, sem_a, sem_s0)
        compute(_NG - 1, idx_r0, idx_c0, idx_rc0, trow0, sem_a, val0, sem_s0)
    pltpu.make_async_copy(val0, spacc.at[idx_r0], sem_s0).wait()
    pltpu.make_async_copy(val1, spacc.at[idx_r1], sem_s1).wait()
    pltpu.sync_copy(s1d, sd_hbm.at[wid])
    plsc.subcore_barrier()
    for t in range(pl.cdiv(_N // _G, 16)):
        u = sid + 16 * t

        @pl.when(u < _N // _G)
        def _drain():
            pltpu.sync_copy(spacc.at[pl.ds(u * _G, _G)],
                            acc_hbm.at[cid, pl.ds(u * _G, _G)])


def _full_spec(shape):
    nd = len(shape)
    return pl.BlockSpec(shape, lambda i: (0,) * nd)


def kernel(x, z, edge_index, edge_weight, W_chi, W_phi, W_varphi):
    row = edge_index[0]
    col = edge_index[1]

    p, pc = pl.pallas_call(
        _pre_body,
        grid=(_N // _BP,),
        in_specs=[
            pl.BlockSpec((_BP, _D), lambda i: (i, 0)),
            _full_spec((_D, _D)),
            _full_spec((_D, _D)),
            _full_spec((_D, _D)),
        ],
        out_specs=[
            pl.BlockSpec((_BP, 2 * _D), lambda i: (i, 0)),
            pl.BlockSpec((_BP, 1), lambda i: (i, 0)),
        ],
        out_shape=[
            jax.ShapeDtypeStruct((_N, 2 * _D), jnp.bfloat16),
            jax.ShapeDtypeStruct((_N, 1), jnp.float32),
        ],
    )(x, W_chi, W_phi, W_varphi)

    acc, sd = pl.kernel(
        _edge_body,
        out_type=[
            jax.ShapeDtypeStruct((2, _N, _D), jnp.float32),
            jax.ShapeDtypeStruct((_NW, 2 * _N), jnp.float32),
        ],
        mesh=plsc.VectorSubcoreMesh(core_axis_name="c", subcore_axis_name="s"),
        compiler_params=pltpu.CompilerParams(needs_layout_passes=False),
        scratch_types=[
            pltpu.VMEM((_SLAB,), jnp.int32),
            pltpu.VMEM((_SLAB,), jnp.int32),
            pltpu.VMEM((_SLAB,), jnp.float32),
            pltpu.VMEM((_G,), jnp.int32),
            pltpu.VMEM((_G,), jnp.int32),
            pltpu.VMEM((_G,), jnp.int32),
            pltpu.VMEM((_G,), jnp.int32),
            pltpu.VMEM((2 * _G,), jnp.int32),
            pltpu.VMEM((2 * _G,), jnp.int32),
            pltpu.VMEM((2 * _G, _D), jnp.int32),
            pltpu.VMEM((2 * _G, _D), jnp.int32),
            pltpu.VMEM((_G, _D), jnp.float32),
            pltpu.VMEM((_G, _D), jnp.float32),
            pltpu.VMEM((2 * _N,), jnp.float32),
            pltpu.VMEM_SHARED((_N, _D), jnp.float32),
            pltpu.SemaphoreType.DMA,
            pltpu.SemaphoreType.DMA,
            pltpu.SemaphoreType.DMA,
            pltpu.SemaphoreType.DMA,
        ],
    )(lax.bitcast_convert_type(p.reshape(_N, _D, 2), jnp.int32),
      z, row, col, edge_weight)

    sdr = sd.reshape(_NW, _N, 2)
    out = pl.pallas_call(
        _post_body,
        grid=(_N // _BP,),
        in_specs=[
            pl.BlockSpec((_BP, _D), lambda i: (i, 0)),
            pl.BlockSpec((_BP, _D), lambda i: (i, 0)),
            pl.BlockSpec((2, _BP, _D), lambda i: (0, i, 0)),
            pl.BlockSpec((_NW, _BP, 2), lambda i: (0, i, 0)),
            pl.BlockSpec((_BP, 1), lambda i: (i, 0)),
        ],
        out_specs=pl.BlockSpec((_BP, _D), lambda i: (i, 0)),
        out_shape=jax.ShapeDtypeStruct((_N, _D), jnp.float32),
    )(x, z, acc, sdr, pc)
    return out


# speedup vs baseline: 1.0262x; 1.0262x over previous
import jax
import jax.numpy as jnp
from jax import lax
from jax.experimental import pallas as pl
from jax.experimental.pallas import tpu as pltpu
from jax.experimental.pallas import tpu_sc as plsc

_N = 10000
_E = 320000
_D = 128
_NW = 32
_EPT = _E // _NW
_G = 16
_SLAB = 2000
_NSLAB = _EPT // _SLAB
_NG = _SLAB // _G
_RPT = _N // 16
_BP = 1000


def _pre_body(x_ref, wc_ref, wp_ref, wv_ref, p_ref, pc_ref):
    x = x_ref[...]
    xc = jnp.dot(x, wc_ref[...].T, preferred_element_type=jnp.float32)
    h = jnp.dot(xc, wp_ref[...].T, preferred_element_type=jnp.float32)
    v = jnp.dot(x, wv_ref[...].T, preferred_element_type=jnp.float32)
    p_ref[:, 0:_D] = h
    p_ref[:, _D:2 * _D] = v
    pc_ref[...] = jnp.tanh(jnp.sqrt(jnp.sum(xc * xc, axis=1, keepdims=True)))


def _post_body(x_ref, z_ref, acc_ref, sd_ref, pc_ref, o_ref):
    s2 = acc_ref[0] + acc_ref[1]
    sd = jnp.sum(sd_ref[...], axis=0)
    s1 = sd[:, 0:1]
    deg = sd[:, 1:2]
    f = jnp.where(deg > 0.0, 1.0 / (deg * pc_ref[...]), 0.0)
    o_ref[...] = x_ref[...] - f * (s1 * z_ref[...] - s2)


def _edge_body(p_hbm, z_hbm, row_hbm, col_hbm, ew_hbm, acc_hbm, sd_hbm,
               row_v, col_v, ew_v, idx_r0, idx_c0, idx_r1, idx_c1,
               idx_rc0, idx_rc1, trow0, trow1, val0, val1,
               s1d, spacc, sem_a, sem_b, sem_s0, sem_s1):
    cid = lax.axis_index("c")
    sid = lax.axis_index("s")
    wid = sid * 2 + cid
    ebase = wid * _EPT

    zeros16 = jnp.zeros((16,), jnp.float32)
    for r in range(_G):
        for j in range(_D // 16):
            val0[r, pl.ds(16 * j, 16)] = zeros16
            val1[r, pl.ds(16 * j, 16)] = zeros16
    izero = jnp.zeros((16,), jnp.int32)
    idx_r0[...] = izero
    idx_r1[...] = izero
    def zclr(i, carry):
        s1d[pl.ds(i * 16, 16)] = zeros16
        return carry
    lax.fori_loop(0, 2 * _N // 16, zclr, 0)
    n_units = _N // _G
    for t in range(pl.cdiv(n_units, 16)):
        u = sid + 16 * t

        @pl.when(u < n_units)
        def _zero():
            pltpu.sync_copy(val0, spacc.at[pl.ds(u * _G, _G)])
    plsc.subcore_barrier()
    pltpu.async_copy(val0, spacc.at[idx_r0], sem_s0, add=True)
    pltpu.async_copy(val1, spacc.at[idx_r1], sem_s1, add=True)

    def issue(g, idx_r, idx_c, idx_rc, trow, val, sem, sem_s):
        pltpu.make_async_copy(val, spacc.at[idx_r], sem_s).wait()
        r16 = row_v[pl.ds(g * _G, _G)]
        c16 = col_v[pl.ds(g * _G, _G)]
        idx_r[...] = r16
        idx_c[...] = c16
        idx_rc[pl.ds(0, _G)] = r16
        idx_rc[pl.ds(_G, _G)] = c16
        pltpu.async_copy(p_hbm.at[idx_rc], trow, sem)
        pltpu.async_copy(z_hbm.at[idx_c], val, sem)

    def compute(g, idx_r, idx_c, idx_rc, trow, sem, val, sem_s):
        pltpu.make_async_copy(p_hbm.at[idx_rc], trow, sem).wait()
        pltpu.make_async_copy(z_hbm.at[idx_c], val, sem).wait()
        lane = lax.iota(jnp.int32, 16)

        def edot(e, dhdv):
            dh, dv = dhdv
            ph = []
            pv = []
            for j in range(_D // 16):
                ph.append(trow[e, pl.ds(16 * j, 16)]
                          * trow[e + _G, pl.ds(16 * j, 16)])
            for j in range(_D // 16, 2 * _D // 16):
                d = (trow[e, pl.ds(16 * j, 16)]
                     - trow[e + _G, pl.ds(16 * j, 16)])
                pv.append(d * d)
            while len(ph) > 1:
                ph = [ph[k] + ph[k + 1] for k in range(0, len(ph), 2)]
                pv = [pv[k] + pv[k + 1] for k in range(0, len(pv), 2)]
            acc = ph[0]
            acc2 = pv[0]
            for s in (8, 4, 2, 1):
                perm = jnp.bitwise_xor(lane, s)
                acc = acc + jnp.take(acc, perm)
                acc2 = acc2 + jnp.take(acc2, perm)
            dh = jnp.where(lane == e, acc, dh)
            dv = jnp.where(lane == e, acc2, dv)
            return dh, dv

        dh, dv = lax.fori_loop(
            0, _G, edot,
            (jnp.zeros((16,), jnp.float32), jnp.zeros((16,), jnp.float32)))
        ewg = ew_v[pl.ds(g * _G, _G)]
        e2 = jnp.exp(jnp.abs(dh) * 2.0)
        t1 = 1.0 - 2.0 / (e2 + 1.0)
        iy = jnp.int32(0x5F3759DF) - lax.shift_right_arithmetic(
            plsc.bitcast(dv, jnp.int32), 1)
        y = plsc.bitcast(iy, jnp.float32)
        for _ in range(3):
            y = y * (1.5 - 0.5 * dv * y * y)
        nv = dv * y
        r2 = 2.0 / (nv + 1e-6)
        e2b = jnp.exp(r2)
        t2 = 1.0 - 2.0 / (e2b + 1.0)
        ug = ewg * t1 * t2
        r2v = idx_r[...] * 2
        plsc.addupdate_scatter(s1d, [r2v], ug)
        plsc.addupdate_scatter(s1d, [r2v + 1], ewg)
        for e in range(_G):
            u = ug[e]
            for j in range(_D // 16):
                val[e, pl.ds(16 * j, 16)] = val[e, pl.ds(16 * j, 16)] * u
        pltpu.async_copy(val, spacc.at[idx_r], sem_s, add=True)

    def pair(gg, carry):
        g0 = gg * 2
        issue(g0 + 1, idx_r1, idx_c1, idx_rc1, trow1, val1, sem_b, sem_s1)
        compute(g0, idx_r0, idx_c0, idx_rc0, trow0, sem_a, val0, sem_s0)

        @pl.when(gg < _NG // 2 - 1)
        def _prefetch():
            issue(g0 + 2, idx_r0, idx_c0, idx_rc0, trow0, val0, sem_a, sem_s0)
        compute(g0 + 1, idx_r1, idx_c1, idx_rc1, trow1, sem_b, val1, sem_s1)
        return carry

    for slab in range(_NSLAB):
        sbase = ebase + slab * _SLAB
        pltpu.sync_copy(row_hbm.at[pl.ds(sbase, _SLAB)], row_v)
        pltpu.sync_copy(col_hbm.at[pl.ds(sbase, _SLAB)], col_v)
        pltpu.sync_copy(ew_hbm.at[pl.ds(sbase, _SLAB)], ew_v)
        issue(0, idx_r0, idx_c0, idx_rc0, trow0, val0, sem_a, sem_s0)
        lax.fori_loop(0, _NG // 2, pair, 0)
        issue(_NG - 1, idx_r0, idx_c0, idx_rc0, trow0, val0, sem_a, sem_s0)
        compute(_NG - 1, idx_r0, idx_c0, idx_rc0, trow0, sem_a, val0, sem_s0)
    pltpu.make_async_copy(val0, spacc.at[idx_r0], sem_s0).wait()
    pltpu.make_async_copy(val1, spacc.at[idx_r1], sem_s1).wait()
    pltpu.sync_copy(s1d, sd_hbm.at[wid])
    plsc.subcore_barrier()
    for t in range(pl.cdiv(_N // _G, 16)):
        u = sid + 16 * t

        @pl.when(u < _N // _G)
        def _drain():
            pltpu.sync_copy(spacc.at[pl.ds(u * _G, _G)],
                            acc_hbm.at[cid, pl.ds(u * _G, _G)])


def _full_spec(shape):
    nd = len(shape)
    return pl.BlockSpec(shape, lambda i: (0,) * nd)


def kernel(x, z, edge_index, edge_weight, W_chi, W_phi, W_varphi):
    row = edge_index[0]
    col = edge_index[1]

    p, pc = pl.pallas_call(
        _pre_body,
        grid=(_N // _BP,),
        in_specs=[
            pl.BlockSpec((_BP, _D), lambda i: (i, 0)),
            _full_spec((_D, _D)),
            _full_spec((_D, _D)),
            _full_spec((_D, _D)),
        ],
        out_specs=[
            pl.BlockSpec((_BP, 2 * _D), lambda i: (i, 0)),
            pl.BlockSpec((_BP, 1), lambda i: (i, 0)),
        ],
        out_shape=[
            jax.ShapeDtypeStruct((_N, 2 * _D), jnp.float32),
            jax.ShapeDtypeStruct((_N, 1), jnp.float32),
        ],
    )(x, W_chi, W_phi, W_varphi)

    acc, sd = pl.kernel(
        _edge_body,
        out_type=[
            jax.ShapeDtypeStruct((2, _N, _D), jnp.float32),
            jax.ShapeDtypeStruct((_NW, 2 * _N), jnp.float32),
        ],
        mesh=plsc.VectorSubcoreMesh(core_axis_name="c", subcore_axis_name="s"),
        compiler_params=pltpu.CompilerParams(needs_layout_passes=False),
        scratch_types=[
            pltpu.VMEM((_SLAB,), jnp.int32),
            pltpu.VMEM((_SLAB,), jnp.int32),
            pltpu.VMEM((_SLAB,), jnp.float32),
            pltpu.VMEM((_G,), jnp.int32),
            pltpu.VMEM((_G,), jnp.int32),
            pltpu.VMEM((_G,), jnp.int32),
            pltpu.VMEM((_G,), jnp.int32),
            pltpu.VMEM((2 * _G,), jnp.int32),
            pltpu.VMEM((2 * _G,), jnp.int32),
            pltpu.VMEM((2 * _G, 2 * _D), jnp.float32),
            pltpu.VMEM((2 * _G, 2 * _D), jnp.float32),
            pltpu.VMEM((_G, _D), jnp.float32),
            pltpu.VMEM((_G, _D), jnp.float32),
            pltpu.VMEM((2 * _N,), jnp.float32),
            pltpu.VMEM_SHARED((_N, _D), jnp.float32),
            pltpu.SemaphoreType.DMA,
            pltpu.SemaphoreType.DMA,
            pltpu.SemaphoreType.DMA,
            pltpu.SemaphoreType.DMA,
        ],
    )(p, z, row, col, edge_weight)

    sdr = sd.reshape(_NW, _N, 2)
    out = pl.pallas_call(
        _post_body,
        grid=(_N // _BP,),
        in_specs=[
            pl.BlockSpec((_BP, _D), lambda i: (i, 0)),
            pl.BlockSpec((_BP, _D), lambda i: (i, 0)),
            pl.BlockSpec((2, _BP, _D), lambda i: (0, i, 0)),
            pl.BlockSpec((_NW, _BP, 2), lambda i: (0, i, 0)),
            pl.BlockSpec((_BP, 1), lambda i: (i, 0)),
        ],
        out_specs=pl.BlockSpec((_BP, _D), lambda i: (i, 0)),
        out_shape=jax.ShapeDtypeStruct((_N, _D), jnp.float32),
    )(x, z, acc, sdr, pc)
    return out
